# Initial kernel scaffold; baseline (speedup 1.0000x reference)
import jax
import jax.numpy as jnp
from jax.experimental import pallas as pl


def kernel(x, points, neighbor_idx, W1, b1, g1, beta1, kernel_points, kp_w, W2, b2, g2, beta2):
    raise NotImplementedError("write your pallas kernel here")


# speedup vs baseline: 1.4337x; 1.4337x over previous
import functools

import jax
import jax.numpy as jnp
from jax import lax
from jax.experimental import pallas as pl
from jax.experimental.pallas import tpu as pltpu
from jax.experimental.pallas import tpu_sc as plsc

NEG_SLOPE = 0.01
EPS = 1e-5
POINT_INFLUENCE = 0.05


def _leaky(a):
    return jnp.where(a >= 0, a, NEG_SLOPE * a)


def _stats(raw, st_ref, i, n_valid, bn):
    rid = i * bn + lax.broadcasted_iota(jnp.int32, (bn, 1), 0)
    rm = jnp.where(rid < n_valid, raw, 0.0)
    c = raw.shape[1]
    s = jnp.sum(rm, axis=0)[None, :]
    sq = jnp.sum(rm * rm, axis=0)[None, :]
    st_ref[...] = jnp.concatenate(
        [s, sq, jnp.zeros((6, c), jnp.float32)], axis=0)[None]


def _rbf(v):
    return v.astype(jnp.bfloat16).astype(jnp.float32)


def _dot(a, b):
    return jnp.dot(_rbf(a), _rbf(b), preferred_element_type=jnp.float32,
                   precision=jax.lax.Precision.HIGHEST)


def _mm_stats_body(x_ref, w_ref, b_ref, raw_ref, st_ref, *, n_valid, bn):
    i = pl.program_id(0)
    raw = _dot(x_ref[...], w_ref[...]) + b_ref[0, :]
    raw_ref[...] = raw
    _stats(raw, st_ref, i, n_valid, bn)


def _fc2_stats_body(w_ref, kpw_ref, w2_ref, b_ref, raw_ref, st_ref,
                    *, n_valid, bn):
    i = pl.program_id(0)
    xr = _dot(w_ref[...], kpw_ref[...])
    raw = _dot(xr, w2_ref[...]) + b_ref[0, :]
    raw_ref[...] = raw
    rid = i * bn + lax.broadcasted_iota(jnp.int32, (bn, 1), 0)
    rm = jnp.where(rid < n_valid, raw, 0.0)
    c = raw.shape[1]
    s = jnp.sum(rm, axis=0)[None, :]
    sq = jnp.sum(rm * rm, axis=0)[None, :]
    st_ref[...] = jnp.concatenate(
        [s, sq, jnp.zeros((6, c), jnp.float32)], axis=0)[None]


def _table_body(r_ref, p_ref, sc_ref, sh_ref, o_ref, *, bn):
    a = _leaky(r_ref[...] * sc_ref[0, :] + sh_ref[0, :])
    a = a.astype(jnp.bfloat16).astype(jnp.float32)
    o_ref[...] = jnp.concatenate(
        [a, p_ref[...], jnp.zeros((bn, 13), jnp.float32)], axis=1)


def _resid_body(x_ref, r_ref, sc_ref, sh_ref, o_ref):
    o_ref[...] = x_ref[...] + _leaky(r_ref[...] * sc_ref[0, :] + sh_ref[0, :])


def kernel(x, points, neighbor_idx, W1, b1, g1, beta1, kernel_points, kp_w,
           W2, b2, g2, beta2):
    N, C_IN = x.shape
    C_MID = W1.shape[1]
    K_N = neighbor_idx.shape[1]
    NKP = kernel_points.shape[0]
    C_OUT = W2.shape[1]

    NW = 32
    P_R = 24
    SG = 3
    NSG = P_R // SG
    KP_PAD = 40
    TBLW = 48
    ROUNDS = -(-N // (NW * P_R))
    CHUNK = ROUNDS * P_R
    NP_ = NW * CHUNK
    BN = 256
    G = NP_ // BN
    CW = NKP * C_MID

    pad = NP_ - N
    x_p = jnp.pad(x, ((0, pad), (0, 0)))
    pts_p = jnp.pad(points, ((0, pad), (0, 0)))
    idx_p = jnp.pad(neighbor_idx.astype(jnp.int32),
                    ((0, pad), (0, KP_PAD - K_N)))
    idxr = idx_p.reshape(NP_ * KP_PAD // (SG * KP_PAD), SG * KP_PAD)

    raw1, st1 = pl.pallas_call(
        functools.partial(_mm_stats_body, n_valid=N, bn=BN),
        grid=(G,),
        in_specs=[pl.BlockSpec((BN, C_IN), lambda i: (i, 0)),
                  pl.BlockSpec((C_IN, C_MID), lambda i: (0, 0)),
                  pl.BlockSpec((1, C_MID), lambda i: (0, 0))],
        out_specs=[pl.BlockSpec((BN, C_MID), lambda i: (i, 0)),
                   pl.BlockSpec((1, 8, C_MID), lambda i: (i, 0, 0))],
        out_shape=[jax.ShapeDtypeStruct((NP_, C_MID), jnp.float32),
                   jax.ShapeDtypeStruct((G, 8, C_MID), jnp.float32)],
    )(x_p, W1, b1.reshape(1, -1))
    mu1 = jnp.sum(st1[:, 0, :], axis=0) / N
    var1 = jnp.sum(st1[:, 1, :], axis=0) / N - mu1 * mu1
    sc1 = (g1 / jnp.sqrt(var1 + EPS)).reshape(1, -1)
    sh1 = (beta1 - mu1 * sc1[0]).reshape(1, -1)

    table = pl.pallas_call(
        functools.partial(_table_body, bn=BN),
        grid=(G,),
        in_specs=[pl.BlockSpec((BN, C_MID), lambda i: (i, 0)),
                  pl.BlockSpec((BN, 3), lambda i: (i, 0)),
                  pl.BlockSpec((1, C_MID), lambda i: (0, 0)),
                  pl.BlockSpec((1, C_MID), lambda i: (0, 0))],
        out_specs=pl.BlockSpec((BN, TBLW), lambda i: (i, 0)),
        out_shape=jax.ShapeDtypeStruct((NP_, TBLW), jnp.float32),
    )(raw1, pts_p, sc1, sh1)

    kpsq = jnp.sum(kernel_points ** 2, axis=-1)
    maxkp = jnp.sqrt(jnp.max(kpsq))
    slack = 0.3 * maxkp * (POINT_INFLUENCE + maxkp) + 2e-3
    bound = (POINT_INFLUENCE + maxkp + slack) ** 2
    kp_b = kernel_points.astype(jnp.bfloat16).astype(jnp.float32)
    kp_aug = jnp.zeros((8, 16), jnp.float32)
    kp_aug = kp_aug.at[0, :NKP].set(kp_b[:, 0])
    kp_aug = kp_aug.at[1, :NKP].set(kp_b[:, 1])
    kp_aug = kp_aug.at[2, :NKP].set(kp_b[:, 2])
    kp_aug = kp_aug.at[3, :NKP].set(kpsq)
    kp_aug = kp_aug.at[4, :].set(bound)

    IDX_RPW = CHUNK * KP_PAD // (SG * KP_PAD)

    mesh = plsc.VectorSubcoreMesh(core_axis_name="c", subcore_axis_name="s")

    @functools.partial(
        pl.kernel, mesh=mesh,
        compiler_params=pltpu.CompilerParams(use_tc_tiling_on_sc=False),
        out_type=jax.ShapeDtypeStruct((NP_, CW), jnp.float32),
        scratch_types=[
            pltpu.VMEM((NSG, SG * KP_PAD), jnp.int32),
            pltpu.VMEM((P_R * KP_PAD, TBLW), jnp.float32),
            pltpu.VMEM((P_R, TBLW), jnp.float32),
            pltpu.VMEM((P_R, CW), jnp.float32),
            pltpu.VMEM((8, 16), jnp.float32),
            pltpu.SemaphoreType.DMA,
        ],
    )
    def sc_weighted(table_h, idx_h, kp_h, out_h,
                    idx_vm, rows_vm, q_vm, out_vm, kp_vm, sem):
        wid = lax.axis_index("s") * 2 + lax.axis_index("c")
        pltpu.sync_copy(kp_h, kp_vm)
        kpx = kp_vm[0, :]
        kpy = kp_vm[1, :]
        kpz = kp_vm[2, :]
        kpsq_v = kp_vm[3, :]
        bnd = kp_vm[4, :][0]

        def round_body(r, carry):
            qbase = wid * CHUNK + r * P_R
            pltpu.sync_copy(idx_h.at[pl.ds(wid * IDX_RPW + r * NSG, NSG)],
                            idx_vm)
            cds = [pltpu.async_copy(
                table_h.at[idx_vm.at[sg]],
                rows_vm.at[pl.ds(sg * SG * KP_PAD, SG * KP_PAD)], sem)
                for sg in range(NSG)]
            pltpu.sync_copy(table_h.at[pl.ds(qbase, P_R)], q_vm)
            for cd in cds:
                cd.wait()

            def point_body(p, carry2):
                qv = q_vm[p, pl.ds(C_MID, 16)]
                rb = p * KP_PAD
                zv = jnp.zeros((16,), jnp.float32)
                for Ki in range(2 * NKP):
                    out_vm[p, pl.ds(Ki * 16, 16)] = zv

                def _round_bf16(v):
                    bi = lax.bitcast_convert_type(v, jnp.int32)
                    bi = ((bi + jnp.int32(0x7FFF) + ((bi >> 16) & 1))
                          & jnp.int32(-65536))
                    return lax.bitcast_convert_type(bi, jnp.float32)

                def neigh(kk, c3):
                    rr = rb + kk
                    cv = rows_vm[rr, pl.ds(C_MID, 16)]
                    dv = cv - qv
                    dx = dv[0]
                    dy = dv[1]
                    dz = dv[2]
                    d2 = dx * dx + dy * dy + dz * dz

                    @pl.when(d2 <= bnd)
                    def _():
                        f0 = rows_vm[rr, pl.ds(0, 16)]
                        f1 = rows_vm[rr, pl.ds(16, 16)]
                        dvb = _round_bf16(dv)
                        sq = jnp.maximum(
                            d2 + kpsq_v
                            - 2.0 * (dvb[0] * kpx + dvb[1] * kpy
                                     + dvb[2] * kpz), 0.0)
                        ib = lax.bitcast_convert_type(sq, jnp.int32)
                        z = lax.bitcast_convert_type(
                            jnp.int32(0x5F3759DF) - (ib >> 1), jnp.float32)
                        z = z * (1.5 - 0.5 * sq * z * z)
                        z = z * (1.5 - 0.5 * sq * z * z)
                        z = z * (1.5 - 0.5 * sq * z * z)
                        y = sq * z
                        infl = _round_bf16(jnp.maximum(
                            0.0, 1.0 - y * (1.0 / POINT_INFLUENCE)))
                        for Ki in range(NKP):
                            sK = infl[Ki]
                            plsc.addupdate(
                                out_vm.at[p, pl.ds(Ki * C_MID, 16)], sK * f0)
                            plsc.addupdate(
                                out_vm.at[p, pl.ds(Ki * C_MID + 16, 16)],
                                sK * f1)

                    return c3

                lax.fori_loop(0, K_N, neigh, 0)
                return carry2

            lax.fori_loop(0, P_R, point_body, 0)
            pltpu.sync_copy(out_vm, out_h.at[pl.ds(qbase, P_R)])
            return carry

        lax.fori_loop(0, ROUNDS, round_body, 0)

    weighted = sc_weighted(table, idxr, kp_aug)

    kpw_flat = kp_w.reshape(NKP * C_MID, C_MID)
    raw2, st2 = pl.pallas_call(
        functools.partial(_fc2_stats_body, n_valid=N, bn=BN),
        grid=(G,),
        in_specs=[pl.BlockSpec((BN, CW), lambda i: (i, 0)),
                  pl.BlockSpec((CW, C_MID), lambda i: (0, 0)),
                  pl.BlockSpec((C_MID, C_OUT), lambda i: (0, 0)),
                  pl.BlockSpec((1, C_OUT), lambda i: (0, 0))],
        out_specs=[pl.BlockSpec((BN, C_OUT), lambda i: (i, 0)),
                   pl.BlockSpec((1, 8, C_OUT), lambda i: (i, 0, 0))],
        out_shape=[jax.ShapeDtypeStruct((NP_, C_OUT), jnp.float32),
                   jax.ShapeDtypeStruct((G, 8, C_OUT), jnp.float32)],
    )(weighted, kpw_flat, W2, b2.reshape(1, -1))
    mu2 = jnp.sum(st2[:, 0, :], axis=0) / N
    var2 = jnp.sum(st2[:, 1, :], axis=0) / N - mu2 * mu2
    sc2 = (g2 / jnp.sqrt(var2 + EPS)).reshape(1, -1)
    sh2 = (beta2 - mu2 * sc2[0]).reshape(1, -1)

    out_p = pl.pallas_call(
        _resid_body,
        grid=(G,),
        in_specs=[pl.BlockSpec((BN, C_OUT), lambda i: (i, 0)),
                  pl.BlockSpec((BN, C_OUT), lambda i: (i, 0)),
                  pl.BlockSpec((1, C_OUT), lambda i: (0, 0)),
                  pl.BlockSpec((1, C_OUT), lambda i: (0, 0))],
        out_specs=pl.BlockSpec((BN, C_OUT), lambda i: (i, 0)),
        out_shape=jax.ShapeDtypeStruct((NP_, C_OUT), jnp.float32),
    )(x_p, raw2, sc2, sh2)
    return out_p[:N]
